# baseline (speedup 1.0000x reference)
import functools

import jax
import jax.numpy as jnp
from jax import lax
from jax.experimental import pallas as pl
from jax.experimental.pallas import tpu as pltpu
from jax.experimental.pallas import tpu_sc as plsc

VOCAB = 1000000
EMBED_DIM = 64
BATCH = 16384
N_FIELDS = 26

NC, NS = 2, 16
NW = NC * NS
B = BATCH * N_FIELDS
B_PER_W = B // NW
IDX_W = 128
NG = B_PER_W // IDX_W
NBUF = 4
NROUND = NG // NBUF
PAIR_ROWS = VOCAB // 2

_mesh = plsc.VectorSubcoreMesh(core_axis_name="c", subcore_axis_name="s")

_scratch = (
    [pltpu.VMEM((NG, IDX_W), jnp.int32),
     pltpu.VMEM((NG, IDX_W), jnp.int32)]
    + [pltpu.VMEM((IDX_W, 2 * EMBED_DIM), jnp.float32) for _ in range(NBUF)]
    + [pltpu.VMEM((IDX_W, EMBED_DIM), jnp.float32) for _ in range(NBUF)]
    + [pltpu.SemaphoreType.DMA for _ in range(2 * NBUF)]
)


@functools.partial(
    pl.kernel,
    out_type=jax.ShapeDtypeStruct((B, EMBED_DIM), jnp.float32),
    mesh=_mesh,
    scratch_types=_scratch,
    compiler_params=pltpu.CompilerParams(use_tc_tiling_on_sc=False,
                                         needs_layout_passes=False),
)
def _sc_gather(table, idx, out, idx_v, par_v, *bufs_and_sems):
    wide = bufs_and_sems[:NBUF]
    outb = bufs_and_sems[NBUF:2 * NBUF]
    gsems = bufs_and_sems[2 * NBUF:3 * NBUF]
    ssems = bufs_and_sems[3 * NBUF:]
    wid = lax.axis_index("s") * NC + lax.axis_index("c")
    base = wid * B_PER_W
    pltpu.sync_copy(idx.at[wid], idx_v)

    iota = lax.iota(jnp.int32, 16)

    def prep(g):
        for j0 in range(0, IDX_W, 16):
            v = idx_v[g, pl.ds(j0, 16)]
            idx_v[g, pl.ds(j0, 16)] = v >> 1
            par_v[g, pl.ds(j0, 16)] = (v & 1) * EMBED_DIM

    pl.loop(0, NG)(prep)

    def fire(g, b):
        pltpu.async_copy(table.at[idx_v.at[g]], wide[b], gsems[b])

    def drain_gather(b):
        pltpu.make_async_copy(table.at[pl.ds(0, IDX_W)], wide[b],
                              gsems[b]).wait()

    def drain_store(b):
        pltpu.make_async_copy(out.at[pl.ds(0, IDX_W)], outb[b],
                              ssems[b]).wait()

    def select(g, b):
        def row(j):
            pj = plsc.load_gather(
                par_v, [jnp.full((16,), g, jnp.int32),
                        jnp.full((16,), j, jnp.int32)])
            for k in range(0, EMBED_DIM, 16):
                vec = plsc.load_gather(
                    wide[b], [jnp.full((16,), j, jnp.int32), pj + (iota + k)])
                outb[b][j, pl.ds(k, 16)] = vec

        pl.loop(0, IDX_W)(row)

    for b in range(NBUF):
        fire(b, b)

    def step(i):
        for b in range(NBUF):
            g = i * NBUF + b
            drain_gather(b)

            @pl.when(g >= NBUF)
            def _():
                drain_store(b)

            select(g, b)
            pltpu.async_copy(outb[b], out.at[pl.ds(base + g * IDX_W, IDX_W)],
                             ssems[b])
            g_next = g + NBUF

            @pl.when(g_next < NG)
            def _():
                fire(g_next, b)

    pl.loop(0, NROUND)(step)
    for b in range(NBUF):
        drain_store(b)


def kernel(context, W):
    table = W.reshape(PAIR_ROWS, 2 * EMBED_DIM)
    idx = context.astype(jnp.int32).reshape(NW, NG, IDX_W)
    out = _sc_gather(table, idx)
    return out.reshape(BATCH, N_FIELDS, EMBED_DIM)
